# Initial kernel scaffold; baseline (speedup 1.0000x reference)
import functools

import jax
import jax.numpy as jnp
from jax import lax
from jax.experimental import pallas as pl
from jax.experimental.pallas import tpu as pltpu
from jax.experimental.pallas import tpu_sc as plsc

BS = 4
N = 10000
E = 160000
D_IN = 128
TYPE_NUM = 16
HID = 64

NPAD = 10240
NW = 32
ROWS_PW = 40
EPW = ROWS_PW * 128
EPAD = NW * EPW
ZPW = NPAD // 16
BN = 2048
NB = 5


def _sc_hist_body(edge3, nvec_hbm, deg_out, cnt_out,
                  src2d, dst2d, ones2d, vmask2d, zero_v, nvec_v,
                  sh_deg, sh_cnt):
    c = lax.axis_index("c")
    s = lax.axis_index("s")
    wid = c * 16 + s
    f32 = jnp.float32

    def zbody(t, carry):
        zero_v[pl.ds(t * 16, 16)] = jnp.zeros((16,), f32)
        return carry
    lax.fori_loop(0, ZPW // 16, zbody, None)
    pltpu.sync_copy(zero_v, sh_deg.at[pl.ds(s * ZPW, ZPW)])
    pltpu.sync_copy(zero_v, sh_cnt.at[pl.ds(s * ZPW, ZPW)])

    pltpu.sync_copy(edge3.at[0, pl.ds(wid * ROWS_PW, ROWS_PW)], src2d)
    pltpu.sync_copy(edge3.at[1, pl.ds(wid * ROWS_PW, ROWS_PW)], dst2d)
    pltpu.sync_copy(nvec_hbm, nvec_v)
    nv = nvec_v[...]

    def fbody(t, carry):
        j = t // 8
        k = (t % 8) * 16
        dv = dst2d[j, pl.ds(k, 16)]
        ones2d[j, pl.ds(k, 16)] = jnp.ones((16,), f32)
        vmask2d[j, pl.ds(k, 16)] = jnp.where(
            dv == nv, jnp.ones((16,), f32), jnp.zeros((16,), f32))
        return carry
    lax.fori_loop(0, ROWS_PW * 8, fbody, None)

    plsc.subcore_barrier()

    def sbody(j, carry):
        pltpu.sync_copy(ones2d.at[j], sh_deg.at[dst2d.at[j]], add=True)
        pltpu.sync_copy(vmask2d.at[j], sh_cnt.at[src2d.at[j]], add=True)
        return carry
    lax.fori_loop(0, ROWS_PW, sbody, None)

    plsc.subcore_barrier()

    pltpu.sync_copy(sh_deg.at[pl.ds(s * ZPW, ZPW)], deg_out.at[c, pl.ds(s * ZPW, ZPW)])
    pltpu.sync_copy(sh_cnt.at[pl.ds(s * ZPW, ZPW)], cnt_out.at[c, pl.ds(s * ZPW, ZPW)])


_sc_hist = functools.partial(
    pl.kernel,
    mesh=plsc.VectorSubcoreMesh(core_axis_name="c", subcore_axis_name="s"),
    out_type=(jax.ShapeDtypeStruct((2, NPAD), jnp.float32),
              jax.ShapeDtypeStruct((2, NPAD), jnp.float32)),
    scratch_types=[
        pltpu.VMEM((ROWS_PW, 128), jnp.int32),
        pltpu.VMEM((ROWS_PW, 128), jnp.int32),
        pltpu.VMEM((ROWS_PW, 128), jnp.float32),
        pltpu.VMEM((ROWS_PW, 128), jnp.float32),
        pltpu.VMEM((ZPW,), jnp.float32),
        pltpu.VMEM((16,), jnp.int32),
        pltpu.VMEM_SHARED((NPAD,), jnp.float32),
        pltpu.VMEM_SHARED((NPAD,), jnp.float32),
    ],
)(_sc_hist_body)


def _tc_body(x_r, c_r, wax_r, wac_r, ba_r, degp_r, cntp_r, toh_r, wg_r,
             wv_r, bv_r, bg_r, d123_r, x123_r, ha_r, val_r, acc_r, sacc_r):
    i = pl.program_id(1)
    f32 = jnp.float32
    xb = x_r[0]

    h = jnp.dot(xb, wax_r[...], preferred_element_type=f32)
    h = h + c_r[0] * wac_r[...] + ba_r[...]
    ha_r[0] = jnp.where(h >= 0, h, 0.01 * h)

    degc = degp_r[0:1, :] + degp_r[1:2, :]
    cnt = cntp_r[0:1, :] + cntp_r[1:2, :]
    w = lax.rsqrt(degc + 1.0) * cnt
    rows = lax.broadcasted_iota(jnp.int32, (BN, 1), 0) + i * BN
    xbm = jnp.where(rows < N, xb, 0.0)
    xs_part = lax.dot_general(w, xbm, (((1,), (0,)), ((), ())),
                              preferred_element_type=f32)
    s_part = jnp.sum(w)

    @pl.when(i == 0)
    def _init():
        acc_r[...] = jnp.zeros_like(acc_r)
        sacc_r[0] = 0.0

    acc_r[0:1, :] = acc_r[0:1, :] + xs_part
    sacc_r[0] = sacc_r[0] + s_part

    @pl.when(i == NB - 1)
    def _finalize():
        xs = acc_r[0:1, :]
        w1 = wg_r[0:TYPE_NUM, :]
        w2 = wg_r[TYPE_NUM:TYPE_NUM + D_IN, :]
        tp = jnp.dot(toh_r[0], w1, preferred_element_type=f32)
        s_val = sacc_r[0]
        dis123 = lax.rsqrt(d123_r[0, 0] + 1.0)
        xsw = jnp.dot(xs, w2, preferred_element_type=f32)
        x123w = jnp.dot(x123_r[0], w2, preferred_element_type=f32)
        out123 = (dis123 * (tp * s_val + xsw)
                  + (dis123 * dis123) * (tp + x123w) + bg_r[...])
        h123 = jnp.where(out123 >= 0, out123, 0.01 * out123)
        val_r[0] = jnp.dot(h123, wv_r[...], preferred_element_type=f32) + bv_r[...]


def _tc_call(x, cond, wax, wac, ba, deg_p, cnt_p, toh3, wg, wv, bv, bg, d123, x123_3):
    return pl.pallas_call(
        _tc_body,
        grid=(BS, NB),
        in_specs=[
            pl.BlockSpec((1, BN, D_IN), lambda b, i: (b, i, 0)),
            pl.BlockSpec((1, BN, 1), lambda b, i: (b, i, 0)),
            pl.BlockSpec((D_IN, HID), lambda b, i: (0, 0)),
            pl.BlockSpec((1, HID), lambda b, i: (0, 0)),
            pl.BlockSpec((1, HID), lambda b, i: (0, 0)),
            pl.BlockSpec((2, BN), lambda b, i: (0, i)),
            pl.BlockSpec((2, BN), lambda b, i: (0, i)),
            pl.BlockSpec((1, 1, TYPE_NUM), lambda b, i: (b, 0, 0)),
            pl.BlockSpec((TYPE_NUM + D_IN, HID), lambda b, i: (0, 0)),
            pl.BlockSpec((HID, 1), lambda b, i: (0, 0)),
            pl.BlockSpec((1, 1), lambda b, i: (0, 0)),
            pl.BlockSpec((1, HID), lambda b, i: (0, 0)),
            pl.BlockSpec(memory_space=pltpu.SMEM),
            pl.BlockSpec((1, 1, D_IN), lambda b, i: (b, 0, 0)),
        ],
        out_specs=[
            pl.BlockSpec((1, BN, HID), lambda b, i: (b, i, 0)),
            pl.BlockSpec((1, 1, 1), lambda b, i: (b, 0, 0)),
        ],
        out_shape=[
            jax.ShapeDtypeStruct((BS, N, HID), jnp.float32),
            jax.ShapeDtypeStruct((BS, 1, 1), jnp.float32),
        ],
        scratch_shapes=[
            pltpu.VMEM((8, 128), jnp.float32),
            pltpu.SMEM((1,), jnp.float32),
        ],
    )(x, cond, wax, wac, ba, deg_p, cnt_p, toh3, wg, wv, bv, bg, d123, x123_3)


def kernel(x, edge_index, condition_state, node_index, type_index,
           W_gcn, b_gcn, W_actor, b_actor, W_val, b_val):
    f32 = jnp.float32
    nidx = jnp.asarray(node_index, jnp.int32)

    pad = jnp.full((2, EPAD - E), NPAD - 1, jnp.int32)
    edge3 = jnp.concatenate([edge_index.astype(jnp.int32), pad],
                            axis=1).reshape(2, EPAD // 128, 128)
    nvec = jnp.full((16,), nidx, jnp.int32)
    deg_p, cnt_p = _sc_hist(edge3, nvec)

    d123 = (deg_p[0, nidx] + deg_p[1, nidx]).reshape(1, 1)
    toh3 = jax.nn.one_hot(type_index, TYPE_NUM, dtype=f32).reshape(BS, 1, TYPE_NUM)
    x123_3 = jnp.take(x, nidx, axis=1).reshape(BS, 1, D_IN)
    wac = W_actor[0:1, :]
    wax = W_actor[1:, :]
    ba = b_actor.reshape(1, HID)
    bg = b_gcn.reshape(1, HID)
    bv = b_val.reshape(1, 1)

    ha, val = _tc_call(x, condition_state, wax, wac, ba, deg_p, cnt_p,
                       toh3, W_gcn, W_val, bv, bg, d123, x123_3)
    return (val.reshape(BS, 1), ha)


# speedup vs baseline: 186.9017x; 186.9017x over previous
import functools

import jax
import jax.numpy as jnp
from jax import lax
from jax.experimental import pallas as pl
from jax.experimental.pallas import tpu as pltpu
from jax.experimental.pallas import tpu_sc as plsc

BS = 4
N = 10000
E = 160000
D_IN = 128
TYPE_NUM = 16
HID = 64

NPAD = 10240
NW = 32
ROWS_PW = 40
EPW = ROWS_PW * 128
EPAD = NW * EPW
ZPW = NPAD // 16
BN = 2048
NB = 5


def _sc_hist_body(edge3, nvec_hbm, deg_out, cnt_out,
                  src2d, dst2d, ones2d, vmask2d, zero_v, nvec_v,
                  sh_deg, sh_cnt):
    c = lax.axis_index("c")
    s = lax.axis_index("s")
    wid = c * 16 + s
    f32 = jnp.float32

    def zbody(t, carry):
        zero_v[pl.ds(t * 16, 16)] = jnp.zeros((16,), f32)
        return carry
    lax.fori_loop(0, ZPW // 16, zbody, None)
    pltpu.sync_copy(zero_v, sh_deg.at[pl.ds(s * ZPW, ZPW)])
    pltpu.sync_copy(zero_v, sh_cnt.at[pl.ds(s * ZPW, ZPW)])

    pltpu.sync_copy(edge3.at[0, pl.ds(wid * ROWS_PW, ROWS_PW)], src2d)
    pltpu.sync_copy(edge3.at[1, pl.ds(wid * ROWS_PW, ROWS_PW)], dst2d)
    pltpu.sync_copy(nvec_hbm, nvec_v)
    nv = nvec_v[...]

    def fbody(t, carry):
        j = t // 8
        k = (t % 8) * 16
        dv = dst2d[j, pl.ds(k, 16)]
        ones2d[j, pl.ds(k, 16)] = jnp.ones((16,), f32)
        vmask2d[j, pl.ds(k, 16)] = jnp.where(
            dv == nv, jnp.ones((16,), f32), jnp.zeros((16,), f32))
        return carry
    lax.fori_loop(0, ROWS_PW * 8, fbody, None)

    plsc.subcore_barrier()

    def sbody(j, carry):
        pltpu.sync_copy(ones2d.at[j], sh_deg.at[dst2d.at[j]], add=True)
        pltpu.sync_copy(vmask2d.at[j], sh_cnt.at[src2d.at[j]], add=True)
        return carry
    lax.fori_loop(0, ROWS_PW, sbody, None)

    plsc.subcore_barrier()

    pltpu.sync_copy(sh_deg.at[pl.ds(s * ZPW, ZPW)], deg_out.at[c, pl.ds(s * ZPW, ZPW)])
    pltpu.sync_copy(sh_cnt.at[pl.ds(s * ZPW, ZPW)], cnt_out.at[c, pl.ds(s * ZPW, ZPW)])


@functools.cache
def _sc_hist():
    return functools.partial(
        pl.kernel,
        mesh=plsc.VectorSubcoreMesh(core_axis_name="c", subcore_axis_name="s"),
        out_type=(jax.ShapeDtypeStruct((2, NPAD), jnp.float32),
                  jax.ShapeDtypeStruct((2, NPAD), jnp.float32)),
        scratch_types=[
            pltpu.VMEM((ROWS_PW, 128), jnp.int32),
            pltpu.VMEM((ROWS_PW, 128), jnp.int32),
            pltpu.VMEM((ROWS_PW, 128), jnp.float32),
            pltpu.VMEM((ROWS_PW, 128), jnp.float32),
            pltpu.VMEM((ZPW,), jnp.float32),
            pltpu.VMEM((16,), jnp.int32),
            pltpu.VMEM_SHARED((NPAD,), jnp.float32),
            pltpu.VMEM_SHARED((NPAD,), jnp.float32),
        ],
    )(_sc_hist_body)


def _tc_body(x_r, c_r, wax_r, wac_r, ba_r, degp_r, cntp_r, toh_r, wg_r,
             wv_r, bv_r, bg_r, d123_r, x123_r, ha_r, val_r, acc_r, sacc_r):
    i = pl.program_id(1)
    f32 = jnp.float32
    xb = x_r[0]

    h = jnp.dot(xb, wax_r[...], preferred_element_type=f32)
    h = h + c_r[0] * wac_r[...] + ba_r[...]
    ha_r[0] = jnp.where(h >= 0, h, 0.01 * h)

    degc = degp_r[0:1, :] + degp_r[1:2, :]
    cnt = cntp_r[0:1, :] + cntp_r[1:2, :]
    w = lax.rsqrt(degc + 1.0) * cnt
    rows = lax.broadcasted_iota(jnp.int32, (BN, 1), 0) + i * BN
    xbm = jnp.where(rows < N, xb, 0.0)
    xs_part = lax.dot_general(w, xbm, (((1,), (0,)), ((), ())),
                              precision=lax.Precision.HIGHEST,
                              preferred_element_type=f32)
    s_part = jnp.sum(w)

    @pl.when(i == 0)
    def _init():
        acc_r[...] = jnp.zeros_like(acc_r)
        sacc_r[0] = 0.0

    acc_r[0:1, :] = acc_r[0:1, :] + xs_part
    sacc_r[0] = sacc_r[0] + s_part

    @pl.when(i == NB - 1)
    def _finalize():
        xs = acc_r[0:1, :]
        w1 = wg_r[0:TYPE_NUM, :]
        w2 = wg_r[TYPE_NUM:TYPE_NUM + D_IN, :]
        hi = lax.Precision.HIGHEST
        tp = jnp.dot(toh_r[0], w1, precision=hi, preferred_element_type=f32)
        s_val = sacc_r[0]
        dis123 = lax.rsqrt(d123_r[0, 0] + 1.0)
        xsw = jnp.dot(xs, w2, precision=hi, preferred_element_type=f32)
        x123w = jnp.dot(x123_r[0], w2, precision=hi, preferred_element_type=f32)
        out123 = (dis123 * (tp * s_val + xsw)
                  + (dis123 * dis123) * (tp + x123w) + bg_r[...])
        h123 = jnp.where(out123 >= 0, out123, 0.01 * out123)
        val_r[0] = (jnp.dot(h123, wv_r[...], precision=hi,
                            preferred_element_type=f32) + bv_r[...])


def _tc_call(x, cond, wax, wac, ba, deg_p, cnt_p, toh3, wg, wv, bv, bg, d123, x123_3):
    return pl.pallas_call(
        _tc_body,
        grid=(BS, NB),
        in_specs=[
            pl.BlockSpec((1, BN, D_IN), lambda b, i: (b, i, 0)),
            pl.BlockSpec((1, BN, 1), lambda b, i: (b, i, 0)),
            pl.BlockSpec((D_IN, HID), lambda b, i: (0, 0)),
            pl.BlockSpec((1, HID), lambda b, i: (0, 0)),
            pl.BlockSpec((1, HID), lambda b, i: (0, 0)),
            pl.BlockSpec((2, BN), lambda b, i: (0, i)),
            pl.BlockSpec((2, BN), lambda b, i: (0, i)),
            pl.BlockSpec((1, 1, TYPE_NUM), lambda b, i: (b, 0, 0)),
            pl.BlockSpec((TYPE_NUM + D_IN, HID), lambda b, i: (0, 0)),
            pl.BlockSpec((HID, 1), lambda b, i: (0, 0)),
            pl.BlockSpec((1, 1), lambda b, i: (0, 0)),
            pl.BlockSpec((1, HID), lambda b, i: (0, 0)),
            pl.BlockSpec(memory_space=pltpu.SMEM),
            pl.BlockSpec((1, 1, D_IN), lambda b, i: (b, 0, 0)),
        ],
        out_specs=[
            pl.BlockSpec((1, BN, HID), lambda b, i: (b, i, 0)),
            pl.BlockSpec((1, 1, 1), lambda b, i: (b, 0, 0)),
        ],
        out_shape=[
            jax.ShapeDtypeStruct((BS, N, HID), jnp.float32),
            jax.ShapeDtypeStruct((BS, 1, 1), jnp.float32),
        ],
        scratch_shapes=[
            pltpu.VMEM((8, 128), jnp.float32),
            pltpu.SMEM((1,), jnp.float32),
        ],
    )(x, cond, wax, wac, ba, deg_p, cnt_p, toh3, wg, wv, bv, bg, d123, x123_3)


def kernel(x, edge_index, condition_state, node_index, type_index,
           W_gcn, b_gcn, W_actor, b_actor, W_val, b_val):
    f32 = jnp.float32
    nidx = jnp.asarray(node_index, jnp.int32)

    pad = jnp.full((2, EPAD - E), NPAD - 1, jnp.int32)
    edge3 = jnp.concatenate([edge_index.astype(jnp.int32), pad],
                            axis=1).reshape(2, EPAD // 128, 128)
    nvec = jnp.full((16,), nidx, jnp.int32)
    deg_p, cnt_p = _sc_hist()(edge3, nvec)

    d123 = (deg_p[0, nidx] + deg_p[1, nidx]).reshape(1, 1)
    toh3 = jax.nn.one_hot(type_index, TYPE_NUM, dtype=f32).reshape(BS, 1, TYPE_NUM)
    x123_3 = jnp.take(x, nidx, axis=1).reshape(BS, 1, D_IN)
    wac = W_actor[0:1, :]
    wax = W_actor[1:, :]
    ba = b_actor.reshape(1, HID)
    bg = b_gcn.reshape(1, HID)
    bv = b_val.reshape(1, 1)

    ha, val = _tc_call(x, condition_state, wax, wac, ba, deg_p, cnt_p,
                       toh3, W_gcn, W_val, bv, bg, d123, x123_3)
    return (val.reshape(BS, 1), ha)
